# baseline (speedup 1.0000x reference)
import functools

import jax
import jax.numpy as jnp
from jax import lax
from jax.experimental import pallas as pl
from jax.experimental.pallas import tpu as pltpu
from jax.experimental.pallas import tpu_sc as plsc

B = 16384
D = 64
NC = 2
NS = 16
NW = NC * NS
BPW = B // NW
IC = 128
NCHUNK = BPW // IC
GROUPS = BPW // 16


def _make_kernel():
    mesh = plsc.VectorSubcoreMesh(core_axis_name="c", subcore_axis_name="s")

    @functools.partial(
        pl.kernel,
        mesh=mesh,
        compiler_params=pltpu.CompilerParams(
            needs_layout_passes=False, use_tc_tiling_on_sc=False),
        out_type=jax.ShapeDtypeStruct((B,), jnp.float32),
        scratch_types=[
            pltpu.VMEM((NCHUNK, IC), jnp.int32),
            pltpu.VMEM((NCHUNK, IC), jnp.int32),
            pltpu.VMEM((NCHUNK, IC), jnp.int32),
            pltpu.VMEM((BPW, D), jnp.float32),
            pltpu.VMEM((BPW, D), jnp.float32),
            pltpu.VMEM((BPW, D), jnp.float32),
            pltpu.VMEM((BPW,), jnp.float32),
            pltpu.SemaphoreType.DMA,
        ],
    )
    def scores(e1_hbm, r_hbm, e2_hbm, ent_hbm, rel_hbm, out_hbm,
               i1_v, ir_v, i2_v, e1r_v, rr_v, e2r_v, out_v, sem):
        wid = lax.axis_index("s") * NC + lax.axis_index("c")
        base = wid * BPW
        rowsel = pl.ds(wid * NCHUNK, NCHUNK)
        pltpu.sync_copy(e1_hbm.at[rowsel], i1_v)
        pltpu.sync_copy(r_hbm.at[rowsel], ir_v)
        pltpu.sync_copy(e2_hbm.at[rowsel], i2_v)
        copies = []
        for k in range(NCHUNK):
            dst = pl.ds(k * IC, IC)
            copies.append(pltpu.async_copy(ent_hbm.at[i1_v.at[k]],
                                           e1r_v.at[dst], sem))
            copies.append(pltpu.async_copy(rel_hbm.at[ir_v.at[k]],
                                           rr_v.at[dst], sem))
            copies.append(pltpu.async_copy(ent_hbm.at[i2_v.at[k]],
                                           e2r_v.at[dst], sem))
        for c in copies:
            c.wait()

        lane = lax.iota(jnp.int32, 16)

        def group(g, _):
            row = g * 16 + lane

            def dstep(dd, acc):
                col = jnp.full((16,), 0, jnp.int32) + dd
                a = plsc.load_gather(e1r_v, [row, col])
                b = plsc.load_gather(rr_v, [row, col])
                c = plsc.load_gather(e2r_v, [row, col])
                return acc + a * b * c

            acc = lax.fori_loop(0, D, dstep, jnp.zeros((16,), jnp.float32))
            s = 1.0 / (1.0 + jnp.exp(-acc))
            s = 1.0 / (1.0 + jnp.exp(-s))
            out_v[pl.ds(g * 16, 16)] = s
            return 0

        lax.fori_loop(0, GROUPS, group, 0)
        pltpu.sync_copy(out_v, out_hbm.at[pl.ds(base, BPW)])

    return scores


_scores = _make_kernel()


def kernel(e1, r, e2, entity_table, relation_table):
    e1 = e1.astype(jnp.int32).reshape(B // IC, IC)
    r = r.astype(jnp.int32).reshape(B // IC, IC)
    e2 = e2.astype(jnp.int32).reshape(B // IC, IC)
    s = _scores(e1, r, e2, entity_table, relation_table)
    return s.reshape(B, 1)


# speedup vs baseline: 1.0429x; 1.0429x over previous
import functools

import jax
import jax.numpy as jnp
from jax import lax
from jax.experimental import pallas as pl
from jax.experimental.pallas import tpu as pltpu
from jax.experimental.pallas import tpu_sc as plsc

B = 16384
D = 64
NC = 2
NS = 16
NW = NC * NS
BPW = B // NW
IC = 128
NCHUNK = BPW // IC
GPC = IC // 16


def _make_kernel():
    mesh = plsc.VectorSubcoreMesh(core_axis_name="c", subcore_axis_name="s")

    @functools.partial(
        pl.kernel,
        mesh=mesh,
        compiler_params=pltpu.CompilerParams(
            needs_layout_passes=False, use_tc_tiling_on_sc=False),
        out_type=jax.ShapeDtypeStruct((B,), jnp.float32),
        scratch_types=[
            pltpu.VMEM((BPW,), jnp.int32),
            pltpu.VMEM((BPW,), jnp.int32),
            pltpu.VMEM((BPW,), jnp.int32),
            pltpu.VMEM((BPW, D), jnp.float32),
            pltpu.VMEM((BPW, D), jnp.float32),
            pltpu.VMEM((BPW, D), jnp.float32),
            pltpu.VMEM((BPW,), jnp.float32),
            pltpu.SemaphoreType.DMA,
            pltpu.SemaphoreType.DMA,
            pltpu.SemaphoreType.DMA,
            pltpu.SemaphoreType.DMA,
        ],
    )
    def scores(e1_hbm, r_hbm, e2_hbm, ent_hbm, rel_hbm, out_hbm,
               i1_v, ir_v, i2_v, e1r_v, rr_v, e2r_v, out_v,
               sem0, sem1, sem2, sem3):
        wid = lax.axis_index("s") * NC + lax.axis_index("c")
        base = wid * BPW
        pltpu.sync_copy(e1_hbm.at[pl.ds(base, BPW)], i1_v)
        pltpu.sync_copy(r_hbm.at[pl.ds(base, BPW)], ir_v)
        pltpu.sync_copy(e2_hbm.at[pl.ds(base, BPW)], i2_v)
        sems = (sem0, sem1, sem2, sem3)
        copies = []
        for k in range(NCHUNK):
            sel = pl.ds(k * IC, IC)
            copies.append((
                pltpu.async_copy(ent_hbm.at[i1_v.at[sel]], e1r_v.at[sel],
                                 sems[k]),
                pltpu.async_copy(rel_hbm.at[ir_v.at[sel]], rr_v.at[sel],
                                 sems[k]),
                pltpu.async_copy(ent_hbm.at[i2_v.at[sel]], e2r_v.at[sel],
                                 sems[k]),
            ))

        lane = lax.iota(jnp.int32, 16)
        zero = jnp.zeros((16,), jnp.float32)

        UNROLL = 8

        def group(g, _):
            row = g * 16 + lane

            def dblk(t, accs):
                accs = list(accs)
                d0 = t * UNROLL
                for u in range(UNROLL):
                    col = jnp.full((16,), 0, jnp.int32) + (d0 + u)
                    a = plsc.load_gather(e1r_v, [row, col])
                    b = plsc.load_gather(rr_v, [row, col])
                    c = plsc.load_gather(e2r_v, [row, col])
                    accs[u % 4] = accs[u % 4] + a * b * c
                return tuple(accs)

            accs = lax.fori_loop(0, D // UNROLL, dblk,
                                 (zero, zero, zero, zero))
            acc = (accs[0] + accs[1]) + (accs[2] + accs[3])
            s = 1.0 / (1.0 + jnp.exp(-acc))
            s = 1.0 / (1.0 + jnp.exp(-s))
            out_v[pl.ds(g * 16, 16)] = s
            return 0

        for k in range(NCHUNK):
            for c in copies[k]:
                c.wait()
            lax.fori_loop(k * GPC, (k + 1) * GPC, group, 0)

        pltpu.sync_copy(out_v, out_hbm.at[pl.ds(base, BPW)])

    return scores


_scores = _make_kernel()


def kernel(e1, r, e2, entity_table, relation_table):
    e1 = e1.astype(jnp.int32)
    r = r.astype(jnp.int32)
    e2 = e2.astype(jnp.int32)
    s = _scores(e1, r, e2, entity_table, relation_table)
    return s.reshape(B, 1)
